# baseline (speedup 1.0000x reference)
import jax
import jax.numpy as jnp
from jax.experimental import pallas as pl


_BLOCK = 8000


def _phi_body(src_ref, e_ref, tgt_ref, out_ref):
    de = e_ref.shape[1]
    d = src_ref.shape[1]
    m1 = jnp.full((de, de), 1.0 / de, jnp.float32)
    g16 = jax.nn.sigmoid(jnp.dot(e_ref[...], m1, preferred_element_type=jnp.float32))
    m2 = jnp.full((de, d), 1.0 / de, jnp.float32)
    gate = jnp.dot(g16, m2, preferred_element_type=jnp.float32)
    out_ref[...] = src_ref[...] * gate + tgt_ref[...]


def kernel(src, e, tgt):
    n, d = src.shape
    de = e.shape[1]
    grid = n // _BLOCK
    return pl.pallas_call(
        _phi_body,
        grid=(grid,),
        in_specs=[
            pl.BlockSpec((_BLOCK, d), lambda i: (i, 0)),
            pl.BlockSpec((_BLOCK, de), lambda i: (i, 0)),
            pl.BlockSpec((_BLOCK, d), lambda i: (i, 0)),
        ],
        out_specs=pl.BlockSpec((_BLOCK, d), lambda i: (i, 0)),
        out_shape=jax.ShapeDtypeStruct((n, d), src.dtype),
    )(src, e, tgt)
